# baseline (speedup 1.0000x reference)
import functools

import jax
import jax.numpy as jnp
from jax import lax
from jax.experimental import pallas as pl
from jax.experimental.pallas import tpu as pltpu
from jax.experimental.pallas import tpu_sc as plsc

B, N, D = 1024, 128, 128
RC = 64
BC = 32

NC, NS = 2, 16
NW = NC * NS
BPW = B // NW
ZWORDS = N * D
L = 16


def _rank_body(rand_ref, ranks_ref):
    r = rand_ref[...]
    rj = r[:, None, :]
    rk = r[:, :, None]
    k_iota = lax.broadcasted_iota(jnp.int32, (RC, N, N), 1)
    j_iota = lax.broadcasted_iota(jnp.int32, (RC, N, N), 2)
    lt = (rk < rj) | ((rk == rj) & (k_iota < j_iota))
    ranks_ref[...] = jnp.sum(lt.astype(jnp.int32), axis=1)


def _col_body(ranks_ref, col_ref):
    ranks = ranks_ref[...]
    n_iota = lax.broadcasted_iota(jnp.int32, (BC, N, N), 1)
    col_ref[...] = (ranks[:, None, :] == n_iota).astype(jnp.float32)


def _sc_row_body(out_hbm, zbuf, sem):
    wid = lax.axis_index("s") * NC + lax.axis_index("c")
    base = wid * BPW * ZWORDS

    def zstep(i, carry):
        zbuf[pl.ds(i * L, L)] = jnp.zeros((L,), jnp.float32)
        return carry

    lax.fori_loop(0, ZWORDS // L, zstep, 0)
    copies = [
        pltpu.async_copy(zbuf, out_hbm.at[pl.ds(base + t * ZWORDS, ZWORDS)], sem)
        for t in range(BPW)
    ]
    for cp in copies:
        cp.wait()


_sc_row = functools.partial(
    pl.kernel,
    out_type=jax.ShapeDtypeStruct((B * N * D,), jnp.float32),
    mesh=plsc.VectorSubcoreMesh(core_axis_name="c", subcore_axis_name="s"),
    scratch_types=[
        pltpu.VMEM((ZWORDS,), jnp.float32),
        pltpu.SemaphoreType.DMA,
    ],
    cost_estimate=pl.CostEstimate(
        flops=1_000_000, bytes_accessed=B * N * D * 4, transcendentals=0
    ),
)(_sc_row_body)


def kernel(distance_matrix):
    rand = jax.random.uniform(jax.random.key(42), (B, N), dtype=jnp.float32)
    row_flat = _sc_row()
    ranks = pl.pallas_call(
        _rank_body,
        grid=(B // RC,),
        in_specs=[pl.BlockSpec((RC, N), lambda i: (i, 0))],
        out_specs=pl.BlockSpec((RC, N), lambda i: (i, 0)),
        out_shape=jax.ShapeDtypeStruct((B, N), jnp.int32),
    )(rand)
    col_emb = pl.pallas_call(
        _col_body,
        grid=(B // BC,),
        in_specs=[pl.BlockSpec((BC, N), lambda i: (i, 0))],
        out_specs=pl.BlockSpec((BC, N, D), lambda i: (i, 0, 0)),
        out_shape=jax.ShapeDtypeStruct((B, N, D), jnp.float32),
    )(ranks)
    return (row_flat.reshape(B, N, D), col_emb, distance_matrix)
